# initial kernel scaffold (unmeasured)
import jax
import jax.numpy as jnp
from jax import lax
from jax.experimental import pallas as pl
from jax.experimental.pallas import tpu as pltpu

N_DEV = 4


def kernel(x, w_mat):
    m, k_per = x.shape
    _, n = w_mat.shape
    chunk = m // N_DEV
    n_steps = N_DEV - 1

    def body(x_ref, w_ref, out_ref, comm_ref, send_sems, recv_sems):
        my = lax.axis_index("i")
        left = lax.rem(my + N_DEV - 1, N_DEV)
        right = lax.rem(my + 1, N_DEV)

        barrier = pltpu.get_barrier_semaphore()
        for nbr in (left, right):
            pl.semaphore_signal(
                barrier, inc=1,
                device_id=(nbr,), device_id_type=pl.DeviceIdType.MESH,
            )
        pl.semaphore_wait(barrier, 2)

        out_ref[:, :] = jnp.dot(
            x_ref[:, :], w_ref[:, :], preferred_element_type=jnp.float32
        )

        for s in range(n_steps):
            c_send = lax.rem(my + N_DEV - s, N_DEV)
            c_recv = lax.rem(my + 2 * N_DEV - s - 1, N_DEV)
            rdma = pltpu.make_async_remote_copy(
                src_ref=out_ref.at[pl.ds(c_send * chunk, chunk)],
                dst_ref=comm_ref.at[s],
                send_sem=send_sems.at[s],
                recv_sem=recv_sems.at[s],
                device_id=(right,),
                device_id_type=pl.DeviceIdType.MESH,
            )
            rdma.start()
            rdma.wait()
            rows = pl.ds(c_recv * chunk, chunk)
            out_ref[rows, :] = out_ref[rows, :] + comm_ref[s]

        for s in range(n_steps):
            a_send = lax.rem(my + 1 + N_DEV - s, N_DEV)
            a_recv = lax.rem(my + N_DEV - s, N_DEV)
            t = n_steps + s
            rdma = pltpu.make_async_remote_copy(
                src_ref=out_ref.at[pl.ds(a_send * chunk, chunk)],
                dst_ref=comm_ref.at[t],
                send_sem=send_sems.at[t],
                recv_sem=recv_sems.at[t],
                device_id=(right,),
                device_id_type=pl.DeviceIdType.MESH,
            )
            rdma.start()
            rdma.wait()
            out_ref[pl.ds(a_recv * chunk, chunk), :] = comm_ref[t]

    return pl.pallas_call(
        body,
        out_shape=jax.ShapeDtypeStruct((m, n), jnp.float32),
        in_specs=[
            pl.BlockSpec(memory_space=pltpu.VMEM),
            pl.BlockSpec(memory_space=pltpu.VMEM),
        ],
        out_specs=pl.BlockSpec(memory_space=pltpu.VMEM),
        scratch_shapes=[
            pltpu.VMEM((2 * n_steps, chunk, n), jnp.float32),
            pltpu.SemaphoreType.DMA((2 * n_steps,)),
            pltpu.SemaphoreType.DMA((2 * n_steps,)),
        ],
        compiler_params=pltpu.CompilerParams(collective_id=0),
    )(x, w_mat)


# baseline (device time: 311375 ns/iter reference)
import jax
import jax.numpy as jnp
from jax import lax
from jax.experimental import pallas as pl
from jax.experimental.pallas import tpu as pltpu

N_DEV = 4


def kernel(x, w_mat):
    m, k_per = x.shape
    _, n = w_mat.shape
    chunk = m // N_DEV
    n_steps = N_DEV - 1

    def body(x_ref, w_ref, out_ref, comm_ref, send_sems, recv_sems):
        my = lax.axis_index("i")
        left = lax.rem(my + N_DEV - 1, N_DEV)
        right = lax.rem(my + 1, N_DEV)

        barrier = pltpu.get_barrier_semaphore()
        for nbr in (left, right):
            pl.semaphore_signal(
                barrier, inc=1,
                device_id=(nbr,), device_id_type=pl.DeviceIdType.MESH,
            )
        pl.semaphore_wait(barrier, 2)

        out_ref[:, :] = jnp.dot(
            x_ref[:, :], w_ref[:, :], preferred_element_type=jnp.float32
        )

        for s in range(n_steps):
            c_send = lax.rem(my + N_DEV - s, N_DEV)
            c_recv = lax.rem(my + 2 * N_DEV - s - 1, N_DEV)
            rdma = pltpu.make_async_remote_copy(
                src_ref=out_ref.at[pl.ds(c_send * chunk, chunk)],
                dst_ref=comm_ref.at[s],
                send_sem=send_sems.at[s],
                recv_sem=recv_sems.at[s],
                device_id=(right,),
                device_id_type=pl.DeviceIdType.MESH,
            )
            rdma.start()
            rdma.wait()
            rows = pl.ds(c_recv * chunk, chunk)
            out_ref[rows, :] = out_ref[rows, :] + comm_ref[s]

        for s in range(n_steps):
            a_send = lax.rem(my + 1 + N_DEV - s, N_DEV)
            a_recv = lax.rem(my + N_DEV - s, N_DEV)
            t = n_steps + s
            rdma = pltpu.make_async_remote_copy(
                src_ref=out_ref.at[pl.ds(a_send * chunk, chunk)],
                dst_ref=comm_ref.at[t],
                send_sem=send_sems.at[t],
                recv_sem=recv_sems.at[t],
                device_id=(right,),
                device_id_type=pl.DeviceIdType.MESH,
            )
            rdma.start()
            rdma.wait()
            out_ref[pl.ds(a_recv * chunk, chunk), :] = comm_ref[t]

    return pl.pallas_call(
        body,
        out_shape=jax.ShapeDtypeStruct((m, n), jnp.float32),
        in_specs=[
            pl.BlockSpec(memory_space=pltpu.VMEM),
            pl.BlockSpec(memory_space=pltpu.VMEM),
        ],
        out_specs=pl.BlockSpec(memory_space=pltpu.VMEM),
        scratch_shapes=[
            pltpu.VMEM((2 * n_steps, chunk, n), jnp.float32),
            pltpu.SemaphoreType.DMA((2 * n_steps,)),
            pltpu.SemaphoreType.DMA((2 * n_steps,)),
        ],
        compiler_params=pltpu.CompilerParams(
            collective_id=0,
            vmem_limit_bytes=100 * 1024 * 1024,
        ),
    )(x, w_mat)


# device time: 172970 ns/iter; 1.8002x vs baseline; 1.8002x over previous
import jax
import jax.numpy as jnp
from jax import lax
from jax.experimental import pallas as pl
from jax.experimental.pallas import tpu as pltpu

N_DEV = 4


def kernel(x, w_mat):
    m, k_per = x.shape
    _, n = w_mat.shape
    chunk = m // N_DEV
    half = n // 2
    n_steps = N_DEV - 1

    def body(x_ref, w_ref, out_ref, comm_ref, send_sems, recv_sems):
        my = lax.axis_index("i")
        left = lax.rem(my + N_DEV - 1, N_DEV)
        right = lax.rem(my + 1, N_DEV)

        barrier = pltpu.get_barrier_semaphore()
        for nbr in (left, right):
            pl.semaphore_signal(
                barrier, inc=1,
                device_id=(nbr,), device_id_type=pl.DeviceIdType.MESH,
            )
        pl.semaphore_wait(barrier, 2)

        def gemm_chunk(c):
            rows = pl.ds(c * chunk, chunk)
            out_ref[rows, :] = jnp.dot(
                x_ref[rows, :], w_ref[:, :],
                preferred_element_type=jnp.float32,
            )

        A = pl.ds(0, half)
        B = pl.ds(half, half)

        def make_pair(slot, rows_a, rows_b, dst_r, dst_l):
            r = pltpu.make_async_remote_copy(
                src_ref=out_ref.at[pl.ds(rows_a * chunk, chunk), A],
                dst_ref=comm_ref.at[slot],
                send_sem=send_sems.at[slot],
                recv_sem=recv_sems.at[slot],
                device_id=(dst_r,),
                device_id_type=pl.DeviceIdType.MESH,
            )
            l = pltpu.make_async_remote_copy(
                src_ref=out_ref.at[pl.ds(rows_b * chunk, chunk), B],
                dst_ref=comm_ref.at[slot + 1],
                send_sem=send_sems.at[slot + 1],
                recv_sem=recv_sems.at[slot + 1],
                device_id=(dst_l,),
                device_id_type=pl.DeviceIdType.MESH,
            )
            return r, l

        gemm_chunk(my)
        rs_r0, rs_l0 = make_pair(0, my, my, right, left)
        rs_r0.start()
        rs_l0.start()

        gemm_chunk(lax.rem(my + N_DEV - 1, N_DEV))
        gemm_chunk(lax.rem(my + 1, N_DEV))
        gemm_chunk(lax.rem(my + 2, N_DEV))

        pending = (rs_r0, rs_l0)
        for s in range(n_steps):
            rd_r, rd_l = pending
            rd_r.wait()
            rd_l.wait()
            ra = lax.rem(my + 2 * N_DEV - s - 1, N_DEV)
            rb = lax.rem(my + s + 1, N_DEV)
            rows_a = pl.ds(ra * chunk, chunk)
            rows_b = pl.ds(rb * chunk, chunk)
            out_ref[rows_a, A] = out_ref[rows_a, A] + comm_ref[2 * s]
            out_ref[rows_b, B] = out_ref[rows_b, B] + comm_ref[2 * s + 1]
            if s + 1 < n_steps:
                nxt_r, nxt_l = make_pair(2 * (s + 1), ra, rb, right, left)
                nxt_r.start()
                nxt_l.start()
                pending = (nxt_r, nxt_l)

        base = 2 * n_steps
        ag_r0, ag_l0 = make_pair(
            base,
            lax.rem(my + 1, N_DEV),
            lax.rem(my + N_DEV - 1, N_DEV),
            right, left,
        )
        ag_r0.start()
        ag_l0.start()
        pending = (ag_r0, ag_l0)
        for s in range(n_steps):
            rd_r, rd_l = pending
            rd_r.wait()
            rd_l.wait()
            ra = lax.rem(my + 2 * N_DEV - s, N_DEV)
            rb = lax.rem(my + s, N_DEV)
            if s + 1 < n_steps:
                nxt_r, nxt_l = make_pair(
                    base + 2 * (s + 1), ra, rb, right, left
                )
                out_ref[pl.ds(ra * chunk, chunk), A] = comm_ref[base + 2 * s]
                out_ref[pl.ds(rb * chunk, chunk), B] = comm_ref[base + 2 * s + 1]
                nxt_r.start()
                nxt_l.start()
                pending = (nxt_r, nxt_l)
            else:
                out_ref[pl.ds(ra * chunk, chunk), A] = comm_ref[base + 2 * s]
                out_ref[pl.ds(rb * chunk, chunk), B] = comm_ref[base + 2 * s + 1]

    n_slots = 4 * n_steps
    return pl.pallas_call(
        body,
        out_shape=jax.ShapeDtypeStruct((m, n), jnp.float32),
        in_specs=[
            pl.BlockSpec(memory_space=pltpu.VMEM),
            pl.BlockSpec(memory_space=pltpu.VMEM),
        ],
        out_specs=pl.BlockSpec(memory_space=pltpu.VMEM),
        scratch_shapes=[
            pltpu.VMEM((n_slots, chunk, half), jnp.float32),
            pltpu.SemaphoreType.DMA((n_slots,)),
            pltpu.SemaphoreType.DMA((n_slots,)),
        ],
        compiler_params=pltpu.CompilerParams(
            collective_id=0,
            vmem_limit_bytes=100 * 1024 * 1024,
        ),
    )(x, w_mat)


# device time: 165078 ns/iter; 1.8862x vs baseline; 1.0478x over previous
import jax
import jax.numpy as jnp
from jax import lax
from jax.experimental import pallas as pl
from jax.experimental.pallas import tpu as pltpu

N_DEV = 4
N_STEPS = N_DEV - 1
N_STREAM = 2


def kernel(x, w_mat):
    m, k_per = x.shape
    _, n = w_mat.shape
    chunk = m // N_DEV
    colw = n // (2 * N_STREAM)

    def body(x_ref, w_ref, out_ref, comm_ref, send_sems, recv_sems):
        my = lax.axis_index("i")
        left = lax.rem(my + N_DEV - 1, N_DEV)
        right = lax.rem(my + 1, N_DEV)
        nbr = (right, left)

        barrier = pltpu.get_barrier_semaphore()
        for b in (left, right):
            pl.semaphore_signal(
                barrier, inc=1,
                device_id=(b,), device_id_type=pl.DeviceIdType.MESH,
            )
        pl.semaphore_wait(barrier, 2)

        def gemm_chunk(c):
            rows = pl.ds(c * chunk, chunk)
            out_ref[rows, :] = jnp.dot(
                x_ref[rows, :], w_ref[:, :],
                preferred_element_type=jnp.float32,
            )

        def cols(d, q):
            return pl.ds((2 * d + q) * colw, colw)

        def slot(phase, s, d, q):
            return (phase * N_STEPS + s) * 2 * N_STREAM + 2 * q + d

        def rs_send_rows(d, s):
            return lax.rem(my + (s if d else 2 * N_DEV - s), N_DEV)

        def rs_recv_rows(d, s):
            return lax.rem(
                my + (s + 1 if d else 2 * N_DEV - s - 1), N_DEV
            )

        def ag_recv_rows(d, s):
            return lax.rem(my + (s if d else 2 * N_DEV - s), N_DEV)

        def start_from_out(phase, s, d, q, rows):
            rd = pltpu.make_async_remote_copy(
                src_ref=out_ref.at[pl.ds(rows * chunk, chunk), cols(d, q)],
                dst_ref=comm_ref.at[slot(phase, s, d, q)],
                send_sem=send_sems.at[slot(phase, s, d, q)],
                recv_sem=recv_sems.at[slot(phase, s, d, q)],
                device_id=(nbr[d],),
                device_id_type=pl.DeviceIdType.MESH,
            )
            rd.start()
            return rd

        def start_forward(s_from, s_to, d, q):
            rd = pltpu.make_async_remote_copy(
                src_ref=comm_ref.at[slot(1, s_from, d, q)],
                dst_ref=comm_ref.at[slot(1, s_to, d, q)],
                send_sem=send_sems.at[slot(1, s_to, d, q)],
                recv_sem=recv_sems.at[slot(1, s_to, d, q)],
                device_id=(nbr[d],),
                device_id_type=pl.DeviceIdType.MESH,
            )
            rd.start()
            return rd

        pending = {}

        gemm_chunk(my)
        for q in range(N_STREAM):
            for d in range(2):
                pending[(d, q)] = start_from_out(0, 0, d, q, my)

        gemm_chunk(lax.rem(my + N_DEV - 1, N_DEV))
        gemm_chunk(lax.rem(my + 1, N_DEV))

        for s in range(N_STEPS):
            for q in range(N_STREAM):
                for d in range(2):
                    pending[(d, q)].wait()
                    rr = rs_recv_rows(d, s)
                    rows = pl.ds(rr * chunk, chunk)
                    out_ref[rows, cols(d, q)] = (
                        out_ref[rows, cols(d, q)]
                        + comm_ref[slot(0, s, d, q)]
                    )
                    if s + 1 < N_STEPS:
                        pending[(d, q)] = start_from_out(
                            0, s + 1, d, q, rr
                        )
                if s == 0 and q == 0:
                    gemm_chunk(lax.rem(my + 2, N_DEV))

        owned = (lax.rem(my + 1, N_DEV), lax.rem(my + N_DEV - 1, N_DEV))
        for q in range(N_STREAM):
            for d in range(2):
                pending[(d, q)] = start_from_out(1, 0, d, q, owned[d])
        for s in range(N_STEPS):
            for q in range(N_STREAM):
                for d in range(2):
                    pending[(d, q)].wait()
                    rr = ag_recv_rows(d, s)
                    if s + 1 < N_STEPS:
                        pending[(d, q)] = start_forward(s, s + 1, d, q)
                    out_ref[pl.ds(rr * chunk, chunk), cols(d, q)] = (
                        comm_ref[slot(1, s, d, q)]
                    )

    n_slots = 2 * N_STEPS * 2 * N_STREAM
    return pl.pallas_call(
        body,
        out_shape=jax.ShapeDtypeStruct((m, n), jnp.float32),
        in_specs=[
            pl.BlockSpec(memory_space=pltpu.VMEM),
            pl.BlockSpec(memory_space=pltpu.VMEM),
        ],
        out_specs=pl.BlockSpec(memory_space=pltpu.VMEM),
        scratch_shapes=[
            pltpu.VMEM((n_slots, chunk, colw), jnp.float32),
            pltpu.SemaphoreType.DMA((n_slots,)),
            pltpu.SemaphoreType.DMA((n_slots,)),
        ],
        compiler_params=pltpu.CompilerParams(
            collective_id=0,
            vmem_limit_bytes=100 * 1024 * 1024,
        ),
    )(x, w_mat)


# device time: 164457 ns/iter; 1.8934x vs baseline; 1.0038x over previous
import jax
import jax.numpy as jnp
from jax import lax
from jax.experimental import pallas as pl
from jax.experimental.pallas import tpu as pltpu

N_DEV = 4
N_STEPS = N_DEV - 1
N_STREAM = 2


def kernel(x, w_mat):
    m, k_per = x.shape
    _, n = w_mat.shape
    chunk = m // N_DEV
    colw = n // (2 * N_STREAM)

    def body(x_ref, w_ref, out_ref, send_sems, recv_sems):
        my = lax.axis_index("i")
        left = lax.rem(my + N_DEV - 1, N_DEV)
        right = lax.rem(my + 1, N_DEV)
        nbr = (right, left)

        barrier = pltpu.get_barrier_semaphore()
        for b in (left, right):
            pl.semaphore_signal(
                barrier, inc=1,
                device_id=(b,), device_id_type=pl.DeviceIdType.MESH,
            )
        pl.semaphore_wait(barrier, 2)

        def cols(d, q):
            return pl.ds((2 * d + q) * colw, colw)

        def slot(phase, s, d, q):
            return (phase * N_STEPS + s) * 2 * N_STREAM + 2 * q + d

        def rs_recv_rows(d, s):
            return lax.rem(
                my + (s + 1 if d else 2 * N_DEV - s - 1), N_DEV
            )

        def ag_recv_rows(d, s):
            return lax.rem(my + (s if d else 2 * N_DEV - s), N_DEV)

        def start(phase, s, d, q, rows):
            ref = out_ref.at[pl.ds(rows * chunk, chunk), cols(d, q)]
            rd = pltpu.make_async_remote_copy(
                src_ref=ref,
                dst_ref=ref,
                send_sem=send_sems.at[slot(phase, s, d, q)],
                recv_sem=recv_sems.at[slot(phase, s, d, q)],
                device_id=(nbr[d],),
                device_id_type=pl.DeviceIdType.MESH,
            )
            rd.start()
            return rd

        def block(rows, d, q):
            return jnp.dot(
                x_ref[pl.ds(rows * chunk, chunk), :],
                w_ref[:, cols(d, q)],
                preferred_element_type=jnp.float32,
            )

        pending = {}
        my_rows = pl.ds(my * chunk, chunk)

        for q in range(N_STREAM):
            for d in range(2):
                out_ref[my_rows, cols(d, q)] = block(my, d, q)
                pending[(d, q)] = start(0, 0, d, q, my)

        for s in range(N_STEPS):
            t = {
                (d, q): block(rs_recv_rows(d, s), d, q)
                for q in range(N_STREAM)
                for d in range(2)
            }
            for q in range(N_STREAM):
                for d in range(2):
                    pending[(d, q)].wait()
                    rr = rs_recv_rows(d, s)
                    rows = pl.ds(rr * chunk, chunk)
                    out_ref[rows, cols(d, q)] = (
                        out_ref[rows, cols(d, q)] + t[(d, q)]
                    )
                    if s + 1 < N_STEPS:
                        pending[(d, q)] = start(0, s + 1, d, q, rr)

        owned = (lax.rem(my + 1, N_DEV), lax.rem(my + N_DEV - 1, N_DEV))
        for q in range(N_STREAM):
            for d in range(2):
                pending[(d, q)] = start(1, 0, d, q, owned[d])
        for s in range(N_STEPS):
            for q in range(N_STREAM):
                for d in range(2):
                    pending[(d, q)].wait()
                    if s + 1 < N_STEPS:
                        pending[(d, q)] = start(
                            1, s + 1, d, q, ag_recv_rows(d, s)
                        )

    n_slots = 2 * N_STEPS * 2 * N_STREAM
    return pl.pallas_call(
        body,
        out_shape=jax.ShapeDtypeStruct((m, n), jnp.float32),
        in_specs=[
            pl.BlockSpec(memory_space=pltpu.VMEM),
            pl.BlockSpec(memory_space=pltpu.VMEM),
        ],
        out_specs=pl.BlockSpec(memory_space=pltpu.VMEM),
        scratch_shapes=[
            pltpu.SemaphoreType.DMA((n_slots,)),
            pltpu.SemaphoreType.DMA((n_slots,)),
        ],
        compiler_params=pltpu.CompilerParams(
            collective_id=0,
            vmem_limit_bytes=100 * 1024 * 1024,
        ),
    )(x, w_mat)


# device time: 157305 ns/iter; 1.9794x vs baseline; 1.0455x over previous
import jax
import jax.numpy as jnp
from jax import lax
from jax.experimental import pallas as pl
from jax.experimental.pallas import tpu as pltpu

N_DEV = 4
N_STEPS = N_DEV - 1
N_STREAM = 2


def kernel(x, w_mat):
    m, k_per = x.shape
    _, n = w_mat.shape
    chunk = m // N_DEV
    colw = n // (2 * N_STREAM)

    def body(x_ref, w_ref, out_ref, acc_ref, send_sems, recv_sems, copy_sems):
        my = lax.axis_index("i")
        left = lax.rem(my + N_DEV - 1, N_DEV)
        right = lax.rem(my + 1, N_DEV)
        nbr = (right, left)

        barrier = pltpu.get_barrier_semaphore()
        for b in (left, right):
            pl.semaphore_signal(
                barrier, inc=1,
                device_id=(b,), device_id_type=pl.DeviceIdType.MESH,
            )
        pl.semaphore_wait(barrier, 2)

        def cols(d, q):
            return pl.ds((2 * d + q) * colw, colw)

        def slot(phase, s, d, q):
            return (phase * N_STEPS + s) * 2 * N_STREAM + 2 * q + d

        def rs_recv_rows(d, s):
            return lax.rem(
                my + (s + 1 if d else 2 * N_DEV - s - 1), N_DEV
            )

        def ag_recv_rows(d, s):
            return lax.rem(my + (s if d else 2 * N_DEV - s), N_DEV)

        def start(phase, s, d, q, rows):
            ref = acc_ref.at[pl.ds(rows * chunk, chunk), cols(d, q)]
            rd = pltpu.make_async_remote_copy(
                src_ref=ref,
                dst_ref=ref,
                send_sem=send_sems.at[slot(phase, s, d, q)],
                recv_sem=recv_sems.at[slot(phase, s, d, q)],
                device_id=(nbr[d],),
                device_id_type=pl.DeviceIdType.MESH,
            )
            rd.start()
            return rd

        copies = []

        def flush(rows, d, q):
            cp = pltpu.make_async_copy(
                acc_ref.at[pl.ds(rows * chunk, chunk), cols(d, q)],
                out_ref.at[pl.ds(rows * chunk, chunk), cols(d, q)],
                copy_sems.at[len(copies)],
            )
            cp.start()
            copies.append(cp)

        def block(rows, d, q):
            return jnp.dot(
                x_ref[pl.ds(rows * chunk, chunk), :],
                w_ref[:, cols(d, q)],
                preferred_element_type=jnp.float32,
            )

        pending = {}
        my_rows = pl.ds(my * chunk, chunk)

        for q in range(N_STREAM):
            for d in range(2):
                acc_ref[my_rows, cols(d, q)] = block(my, d, q)
                pending[(d, q)] = start(0, 0, d, q, my)

        for s in range(N_STEPS):
            t = {
                (d, q): block(rs_recv_rows(d, s), d, q)
                for q in range(N_STREAM)
                for d in range(2)
            }
            for q in range(N_STREAM):
                for d in range(2):
                    pending[(d, q)].wait()
                    rr = rs_recv_rows(d, s)
                    rows = pl.ds(rr * chunk, chunk)
                    acc_ref[rows, cols(d, q)] = (
                        acc_ref[rows, cols(d, q)] + t[(d, q)]
                    )
                    if s + 1 < N_STEPS:
                        pending[(d, q)] = start(0, s + 1, d, q, rr)
                    else:
                        pending[(d, q)] = start(1, 0, d, q, rr)
                        flush(rr, d, q)

        for s in range(N_STEPS):
            for q in range(N_STREAM):
                for d in range(2):
                    pending[(d, q)].wait()
                    rr = ag_recv_rows(d, s)
                    if s + 1 < N_STEPS:
                        pending[(d, q)] = start(1, s + 1, d, q, rr)
                    flush(rr, d, q)

        for cp in copies:
            cp.wait()

    n_slots = 2 * N_STEPS * 2 * N_STREAM
    return pl.pallas_call(
        body,
        out_shape=jax.ShapeDtypeStruct((m, n), jnp.float32),
        in_specs=[
            pl.BlockSpec(memory_space=pltpu.VMEM),
            pl.BlockSpec(memory_space=pltpu.VMEM),
        ],
        out_specs=pl.BlockSpec(memory_space=pl.ANY),
        scratch_shapes=[
            pltpu.VMEM((m, n), jnp.float32),
            pltpu.SemaphoreType.DMA((n_slots,)),
            pltpu.SemaphoreType.DMA((n_slots,)),
            pltpu.SemaphoreType.DMA((16,)),
        ],
        compiler_params=pltpu.CompilerParams(
            collective_id=0,
            vmem_limit_bytes=100 * 1024 * 1024,
        ),
    )(x, w_mat)
